# Initial kernel scaffold; baseline (speedup 1.0000x reference)
import jax
import jax.numpy as jnp
from jax import lax
from jax.experimental import pallas as pl
from jax.experimental.pallas import tpu as pltpu
from jax.experimental.pallas import tpu_sc as plsc

NU = 50000
NN = 100000
D = 32
NL = 3
E = 1600000
B = 16384

NC = 2
NS = 16
HALF = NN // NC
RPT = HALF // NS
EC = 1024
CR = EC // 128
NCH = 98
EPT = EC * NCH
EPAD = EPT * NS
ER = EPAD // 128


def _mesh():
    return plsc.VectorSubcoreMesh(
        core_axis_name="c", subcore_axis_name="s",
        num_cores=NC, num_subcores=NS)


def _layer_body(emb, srcr, dstr, wr, out, srcb, dstb, wrawb, dstlb, wflat,
                rows, acc, sem):
    c = lax.axis_index("c")
    s = lax.axis_index("s")
    lo = c * HALF

    zero = jnp.zeros((16,), jnp.float32)

    def zrow(i, _):
        rows[i, pl.ds(0, 16)] = zero
        rows[i, pl.ds(16, 16)] = zero
        return 0

    lax.fori_loop(0, EC, zrow, 0)
    for k in range(RPT // EC):
        pltpu.sync_copy(rows, acc.at[pl.ds(s * RPT + k * EC, EC)])
    rem = RPT % EC
    pltpu.sync_copy(rows.at[pl.ds(0, rem)],
                    acc.at[pl.ds(s * RPT + (RPT // EC) * EC, rem)])
    plsc.subcore_barrier()

    def chunk(ch, _):
        base = (s * NCH + ch) * CR
        pltpu.sync_copy(srcr.at[pl.ds(base, CR)], srcb)
        pltpu.sync_copy(dstr.at[pl.ds(base, CR)], dstb)
        pltpu.sync_copy(wr.at[pl.ds(base, CR)], wrawb)

        def prep(g, _):
            r = g >> 3
            col = (g & 7) * 16
            dv = dstb[r, pl.ds(col, 16)]
            wv = wrawb[r, pl.ds(col, 16)]
            dl = dv - lo
            inb = (dl >= 0) & (dl < HALF)
            dstlb[r, pl.ds(col, 16)] = jnp.where(inb, dl, 0)
            wflat[pl.ds(g * 16, 16)] = jnp.where(inb, wv, 0.0)
            return 0

        lax.fori_loop(0, EC // 16, prep, 0)
        pltpu.async_copy(emb.at[srcb], rows, sem).wait()

        def scale(i, _):
            wv = jnp.full((16,), wflat[i], jnp.float32)
            rows[i, pl.ds(0, 16)] = rows[i, pl.ds(0, 16)] * wv
            rows[i, pl.ds(16, 16)] = rows[i, pl.ds(16, 16)] * wv
            return 0

        lax.fori_loop(0, EC, scale, 0)
        pltpu.sync_copy(rows, acc.at[dstlb], add=True)
        return 0

    lax.fori_loop(0, NCH, chunk, 0)
    plsc.subcore_barrier()
    pltpu.sync_copy(acc.at[pl.ds(s * RPT, RPT)],
                    out.at[pl.ds(lo + s * RPT, RPT)])


def _final_body(e0, e1, e2, e3, usr, itm, uo, io, idxb, r0, r1, r2, r3, sem):
    c = lax.axis_index("c")
    s = lax.axis_index("s")
    wid = s * NC + c
    rpt = B // (NC * NS)
    base = wid * (rpt // 128)

    for ids_hbm, out_hbm, off in ((usr, uo, 0), (itm, io, NU)):
        pltpu.sync_copy(ids_hbm.at[pl.ds(base, rpt // 128)], idxb)
        if off:
            def addoff(g, _):
                r = g >> 3
                col = (g & 7) * 16
                idxb[r, pl.ds(col, 16)] = idxb[r, pl.ds(col, 16)] + off
                return 0
            lax.fori_loop(0, rpt // 16, addoff, 0)
        pltpu.async_copy(e0.at[idxb], r0, sem).wait()
        pltpu.async_copy(e1.at[idxb], r1, sem).wait()
        pltpu.async_copy(e2.at[idxb], r2, sem).wait()
        pltpu.async_copy(e3.at[idxb], r3, sem).wait()

        def avg(i, _):
            for h in (0, 16):
                v = (r0[i, pl.ds(h, 16)] + r1[i, pl.ds(h, 16)]
                     + r2[i, pl.ds(h, 16)] + r3[i, pl.ds(h, 16)]) * 0.25
                r0[i, pl.ds(h, 16)] = v
            return 0

        lax.fori_loop(0, rpt, avg, 0)
        pltpu.sync_copy(r0, out_hbm.at[pl.ds(wid * rpt, rpt)])


def kernel(users, items, edge_index, edge_weight, user_emb, item_emb):
    pad = EPAD - E
    src = jnp.concatenate(
        [edge_index[0], jnp.zeros((pad,), jnp.int32)]).reshape(ER, 128)
    dst = jnp.concatenate(
        [edge_index[1], jnp.zeros((pad,), jnp.int32)]).reshape(ER, 128)
    w = jnp.concatenate(
        [edge_weight, jnp.zeros((pad,), jnp.float32)]).reshape(ER, 128)
    e0 = jnp.concatenate([user_emb, item_emb], axis=0)

    layer = pl.kernel(
        _layer_body,
        out_type=jax.ShapeDtypeStruct((NN, D), jnp.float32),
        mesh=_mesh(),
        scratch_types=[
            pltpu.VMEM((CR, 128), jnp.int32),
            pltpu.VMEM((CR, 128), jnp.int32),
            pltpu.VMEM((CR, 128), jnp.float32),
            pltpu.VMEM((CR, 128), jnp.int32),
            pltpu.VMEM((EC,), jnp.float32),
            pltpu.VMEM((EC, D), jnp.float32),
            pltpu.VMEM_SHARED((HALF, D), jnp.float32),
            pltpu.SemaphoreType.DMA,
        ],
    )
    e1 = layer(e0, src, dst, w)
    e2 = layer(e1, src, dst, w)
    e3 = layer(e2, src, dst, w)

    rpt = B // (NC * NS)
    fin = pl.kernel(
        _final_body,
        out_type=(jax.ShapeDtypeStruct((B, D), jnp.float32),
                  jax.ShapeDtypeStruct((B, D), jnp.float32)),
        mesh=_mesh(),
        scratch_types=[
            pltpu.VMEM((rpt // 128, 128), jnp.int32),
            pltpu.VMEM((rpt, D), jnp.float32),
            pltpu.VMEM((rpt, D), jnp.float32),
            pltpu.VMEM((rpt, D), jnp.float32),
            pltpu.VMEM((rpt, D), jnp.float32),
            pltpu.SemaphoreType.DMA,
        ],
    )
    uo, io = fin(e0, e1, e2, e3,
                 users.reshape(B // 128, 128), items.reshape(B // 128, 128))
    return uo, io


# speedup vs baseline: 6.8604x; 6.8604x over previous
import jax
import jax.numpy as jnp
from jax import lax
from jax.experimental import pallas as pl
from jax.experimental.pallas import tpu as pltpu
from jax.experimental.pallas import tpu_sc as plsc

NU = 50000
NN = 100000
D = 32
NL = 3
E = 1600000
B = 16384

NC = 2
NS = 16
HALF = NN // NC
RPT = HALF // NS
EC = 512
NCH = 196
EPT = EC * NCH
EPAD = EPT * NS


def _mesh():
    return plsc.VectorSubcoreMesh(
        core_axis_name="c", subcore_axis_name="s",
        num_cores=NC, num_subcores=NS)


def _layer_body(emb, srcr, dstr, wr, out, srcb, dstb, wrawb, dstlb, wflat2,
                rows, acc, sem):
    c = lax.axis_index("c")
    s = lax.axis_index("s")
    lo = c * HALF

    zero = jnp.zeros((16,), jnp.float32)

    def zrow(i, _):
        rows[i, pl.ds(0, 16)] = zero
        rows[i, pl.ds(16, 16)] = zero
        return 0

    lax.fori_loop(0, EC, zrow, 0)
    base = s * 3200

    def zcopy(k, _):
        pltpu.sync_copy(rows.at[pl.ds(0, 400)],
                        acc.at[pl.ds(base + k * 400, 400)])
        return 0

    lax.fori_loop(0, 5, zcopy, 0)

    @pl.when(s < NS - 1)
    def _zero_tail():
        lax.fori_loop(5, 8, zcopy, 0)

    plsc.subcore_barrier()

    def chunk(ch, _):
        base = (s * NCH + ch) * EC
        pltpu.sync_copy(srcr.at[pl.ds(base, EC)], srcb)
        pltpu.sync_copy(dstr.at[pl.ds(base, EC)], dstb)
        pltpu.sync_copy(wr.at[pl.ds(base, EC)], wrawb)

        def prep(g, _):
            dv = dstb[pl.ds(g * 16, 16)]
            wv = wrawb[pl.ds(g * 16, 16)]
            dl = dv - lo
            inb = (dl >= 0) & (dl < HALF)
            dstlb[pl.ds(g * 16, 16)] = jnp.where(inb, dl, 0)
            wflat2[g] = jnp.where(inb, wv, 0.0)
            return 0

        lax.fori_loop(0, EC // 16, prep, 0)
        pltpu.async_copy(emb.at[srcb], rows, sem).wait()

        def scale(g, _):
            wv16 = wflat2[g]
            for j in range(16):
                i = g * 16 + j
                wv = jnp.full((16,), wv16[j])
                rows[i, pl.ds(0, 16)] = rows[i, pl.ds(0, 16)] * wv
                rows[i, pl.ds(16, 16)] = rows[i, pl.ds(16, 16)] * wv
            return 0

        lax.fori_loop(0, EC // 16, scale, 0)
        pltpu.sync_copy(rows, acc.at[dstlb], add=True)
        return 0

    lax.fori_loop(0, NCH, chunk, 0)
    plsc.subcore_barrier()

    def wbcopy(k, _):
        pltpu.sync_copy(acc.at[pl.ds(base + k * 400, 400)],
                        out.at[pl.ds(lo + base + k * 400, 400)])
        return 0

    lax.fori_loop(0, 5, wbcopy, 0)

    @pl.when(s < NS - 1)
    def _wb_tail():
        lax.fori_loop(5, 8, wbcopy, 0)


def _final_body(e0, e1, e2, e3, usr, itm, uo, io, idxb, r0, r1, r2, r3, sem):
    c = lax.axis_index("c")
    s = lax.axis_index("s")
    wid = s * NC + c
    rpt = B // (NC * NS)

    for ids_hbm, out_hbm, off in ((usr, uo, 0), (itm, io, NU)):
        pltpu.sync_copy(ids_hbm.at[pl.ds(wid * rpt, rpt)], idxb)
        if off:
            def addoff(g, _):
                idxb[pl.ds(g * 16, 16)] = idxb[pl.ds(g * 16, 16)] + off
                return 0
            lax.fori_loop(0, rpt // 16, addoff, 0)
        pltpu.async_copy(e0.at[idxb], r0, sem).wait()
        pltpu.async_copy(e1.at[idxb], r1, sem).wait()
        pltpu.async_copy(e2.at[idxb], r2, sem).wait()
        pltpu.async_copy(e3.at[idxb], r3, sem).wait()

        def avg(i, _):
            for h in (0, 16):
                v = (r0[i, pl.ds(h, 16)] + r1[i, pl.ds(h, 16)]
                     + r2[i, pl.ds(h, 16)] + r3[i, pl.ds(h, 16)]) * 0.25
                r0[i, pl.ds(h, 16)] = v
            return 0

        lax.fori_loop(0, rpt, avg, 0)
        pltpu.sync_copy(r0, out_hbm.at[pl.ds(wid * rpt, rpt)])


def kernel(users, items, edge_index, edge_weight, user_emb, item_emb):
    pad = EPAD - E
    src = jnp.concatenate([edge_index[0], jnp.zeros((pad,), jnp.int32)])
    dst = jnp.concatenate([edge_index[1], jnp.zeros((pad,), jnp.int32)])
    w = jnp.concatenate([edge_weight, jnp.zeros((pad,), jnp.float32)])
    e0 = jnp.concatenate([user_emb, item_emb], axis=0)

    cp = pltpu.CompilerParams(use_tc_tiling_on_sc=False)
    layer = pl.kernel(
        _layer_body,
        out_type=jax.ShapeDtypeStruct((NN, D), jnp.float32),
        mesh=_mesh(),
        compiler_params=cp,
        scratch_types=[
            pltpu.VMEM((EC,), jnp.int32),
            pltpu.VMEM((EC,), jnp.int32),
            pltpu.VMEM((EC,), jnp.float32),
            pltpu.VMEM((EC,), jnp.int32),
            pltpu.VMEM((EC // 16, 16), jnp.float32),
            pltpu.VMEM((EC, D), jnp.float32),
            pltpu.VMEM_SHARED((HALF, D), jnp.float32),
            pltpu.SemaphoreType.DMA,
        ],
    )
    e1 = layer(e0, src, dst, w)
    e2 = layer(e1, src, dst, w)
    e3 = layer(e2, src, dst, w)

    rpt = B // (NC * NS)
    fin = pl.kernel(
        _final_body,
        out_type=(jax.ShapeDtypeStruct((B, D), jnp.float32),
                  jax.ShapeDtypeStruct((B, D), jnp.float32)),
        mesh=_mesh(),
        compiler_params=cp,
        scratch_types=[
            pltpu.VMEM((rpt,), jnp.int32),
            pltpu.VMEM((rpt, D), jnp.float32),
            pltpu.VMEM((rpt, D), jnp.float32),
            pltpu.VMEM((rpt, D), jnp.float32),
            pltpu.VMEM((rpt, D), jnp.float32),
            pltpu.SemaphoreType.DMA,
        ],
    )
    uo, io = fin(e0, e1, e2, e3, users, items)
    return uo, io
